# baseline (speedup 1.0000x reference)
import functools

import jax
import jax.numpy as jnp
from jax.experimental import pallas as pl

_SPARSITY_RATIO = 0.3
_INT32_MIN = -2147483648


def _topk_mask_body(x_ref, o_ref, *, k):
    x = x_ref[...]
    bits = jax.lax.bitcast_convert_type(x, jnp.int32)
    key = bits ^ ((bits >> 31) & jnp.int32(0x7FFFFFFF))

    ones = jnp.ones((x.shape[1], 1), jnp.bfloat16)
    kf = jnp.float32(k)

    def count_ge(cand):
        m = (key >= cand).astype(jnp.bfloat16)
        return jax.lax.dot_general(
            m, ones, (((1,), (0,)), ((), ())),
            preferred_element_type=jnp.float32)

    cnt = count_ge(jnp.int32(0))
    t = jnp.where(cnt >= kf, jnp.int32(0), jnp.int32(_INT32_MIN))
    for b in range(30, -1, -1):
        cand = t | jnp.int32(1 << b)
        cnt = count_ge(cand)
        t = jnp.where(cnt >= kf, cand, t)

    o_ref[...] = jnp.where(key >= t, x, jnp.float32(0.0))


def kernel(attn_weights):
    shape = attn_weights.shape
    S = shape[-1]
    k = int(S * (1.0 - _SPARSITY_RATIO))
    if k <= 0:
        return attn_weights
    x = attn_weights.reshape(-1, S)
    n = x.shape[0]
    R = 256 if n % 256 == 0 else n
    out = pl.pallas_call(
        functools.partial(_topk_mask_body, k=k),
        grid=(n // R,),
        in_specs=[pl.BlockSpec((R, S), lambda i: (i, 0))],
        out_specs=pl.BlockSpec((R, S), lambda i: (i, 0)),
        out_shape=jax.ShapeDtypeStruct((n, S), x.dtype),
    )(x)
    return out.reshape(shape)


# speedup vs baseline: 2.1081x; 2.1081x over previous
import functools

import jax
import jax.numpy as jnp
from jax.experimental import pallas as pl

_SPARSITY_RATIO = 0.3
_INT32_MIN = -2147483648


def _topk_mask_body(x_ref, o_ref, *, k):
    x = x_ref[...]
    S = x.shape[1]
    bits = jax.lax.bitcast_convert_type(x, jnp.int32)
    key = bits ^ ((bits >> 31) & jnp.int32(0x7FFFFFFF))

    hi = (key >> 16).astype(jnp.int16)
    lo = ((key & jnp.int32(0xFFFF)) - jnp.int32(32768)).astype(jnp.int16)

    def count16(mask):
        m = mask.astype(jnp.int16)
        acc = m[:, 0:256]
        for j in range(1, S // 256):
            acc = acc + m[:, j * 256:(j + 1) * 256]
        return jnp.sum(acc.astype(jnp.int32), axis=1, keepdims=True)

    cnt = count16(hi >= 0)
    ta = jnp.where(cnt >= k, jnp.int32(0), jnp.int32(-32768))
    for b in range(14, -1, -1):
        cand = ta | jnp.int32(1 << b)
        cnt = count16(hi >= cand.astype(jnp.int16))
        ta = jnp.where(cnt >= k, cand, ta)
    ta16 = ta.astype(jnp.int16)

    kb = k - count16(hi > ta16)
    mb = hi == ta16
    cnt = count16(mb & (lo >= 0))
    tb = jnp.where(cnt >= kb, jnp.int32(0), jnp.int32(-32768))
    for b in range(14, -1, -1):
        cand = tb | jnp.int32(1 << b)
        cnt = count16(mb & (lo >= cand.astype(jnp.int16)))
        tb = jnp.where(cnt >= kb, cand, tb)

    t32 = (ta << 16) | (tb + jnp.int32(32768))
    o_ref[...] = jnp.where(key >= t32, x, jnp.float32(0.0))


def kernel(attn_weights):
    shape = attn_weights.shape
    S = shape[-1]
    k = int(S * (1.0 - _SPARSITY_RATIO))
    if k <= 0:
        return attn_weights
    x = attn_weights.reshape(-1, S)
    n = x.shape[0]
    R = 256 if n % 256 == 0 else n
    out = pl.pallas_call(
        functools.partial(_topk_mask_body, k=k),
        grid=(n // R,),
        in_specs=[pl.BlockSpec((R, S), lambda i: (i, 0))],
        out_specs=pl.BlockSpec((R, S), lambda i: (i, 0)),
        out_shape=jax.ShapeDtypeStruct((n, S), x.dtype),
    )(x)
    return out.reshape(shape)
